# Initial kernel scaffold; baseline (speedup 1.0000x reference)
import jax
import jax.numpy as jnp
from jax.experimental import pallas as pl


def kernel(feature_idx, feature_values, W_first, W_second):
    raise NotImplementedError("write your pallas kernel here")


# speedup vs baseline: 1.9624x; 1.9624x over previous
import functools

import jax
import jax.numpy as jnp
from jax import lax
from jax.experimental import pallas as pl
from jax.experimental.pallas import tpu as pltpu
from jax.experimental.pallas import tpu_sc as plsc

B = 16384
F = 26
K = 32
NC = 2
NS = 16
NW = NC * NS
SPW = B // NW
C = 64
NCHUNK = SPW // C
RPC = C * F
G = 128
NG = RPC // G


def _fm_body(idx_hbm, vals_hbm, w1_hbm, w2_hbm, first_hbm, second_hbm,
             idx_v, vals_v, w1_v, rows_v, dmat_v, fmat_v, first_v, second_v,
             sem, sem2):
    wid = lax.axis_index("s") * NC + lax.axis_index("c")
    lane = lax.iota(jnp.int32, 16)
    idx16 = lane * 16
    hi_mask = (lane >= 6).astype(jnp.float32)

    def chunk_body(c, _):
        row_base = (wid * NCHUNK + c) * RPC
        pltpu.sync_copy(idx_hbm.at[pl.ds(row_base, RPC)], idx_v)
        pltpu.sync_copy(vals_hbm.at[pl.ds(row_base, RPC)], vals_v)

        copies = []
        for g in range(NG):
            sl = pl.ds(g * G, G)
            copies.append(pltpu.async_copy(
                w2_hbm.at[idx_v.at[sl]], rows_v.at[sl], sem))
            copies.append(pltpu.async_copy(
                w1_hbm.at[idx_v.at[sl]], w1_v.at[sl], sem2))
        for cp in copies:
            cp.wait()

        def group_body(g, _):
            b0 = g * 16
            for j in range(16):
                r0 = (b0 + j) * F
                v_lo = vals_v[pl.ds(r0, 16)]
                v_hi = vals_v[pl.ds(r0 + 10, 16)]
                w_lo = w1_v[pl.ds(r0, 16)]
                w_hi = w1_v[pl.ds(r0 + 10, 16)]
                acc0 = jnp.zeros((16,), jnp.float32)
                acc1 = jnp.zeros((16,), jnp.float32)
                sq0 = jnp.zeros((16,), jnp.float32)
                sq1 = jnp.zeros((16,), jnp.float32)
                for f in range(F):
                    vf = v_lo[f] if f < 16 else v_hi[f - 10]
                    x0 = rows_v[r0 + f, 0:16]
                    x1 = rows_v[r0 + f, 16:32]
                    t0 = x0 * vf
                    t1 = x1 * vf
                    acc0 = acc0 + t0
                    acc1 = acc1 + t1
                    sq0 = sq0 + t0 * t0
                    sq1 = sq1 + t1 * t1
                d = acc0 * acc0 - sq0 + acc1 * acc1 - sq1
                fv = v_lo * w_lo + (v_hi * w_hi) * hi_mask
                dmat_v[pl.ds(j * 16, 16)] = d
                fmat_v[pl.ds(j * 16, 16)] = fv
            dsum = jnp.zeros((16,), jnp.float32)
            fsum = jnp.zeros((16,), jnp.float32)
            for k in range(16):
                col = idx16 + k
                dsum = dsum + plsc.load_gather(dmat_v, [col])
                fsum = fsum + plsc.load_gather(fmat_v, [col])
            second_v[pl.ds(b0, 16)] = 0.5 * dsum
            first_v[pl.ds(b0, 16)] = fsum
            return 0

        lax.fori_loop(0, C // 16, group_body, 0)

        out_sl = pl.ds(wid * SPW + c * C, C)
        pltpu.sync_copy(first_v, first_hbm.at[out_sl])
        pltpu.sync_copy(second_v, second_hbm.at[out_sl])
        return 0

    lax.fori_loop(0, NCHUNK, chunk_body, 0)


_fm = functools.partial(
    pl.kernel,
    out_type=(jax.ShapeDtypeStruct((B,), jnp.float32),
              jax.ShapeDtypeStruct((B,), jnp.float32)),
    mesh=plsc.VectorSubcoreMesh(core_axis_name="c", subcore_axis_name="s"),
    scratch_types=[
        pltpu.VMEM((RPC,), jnp.int32),
        pltpu.VMEM((RPC,), jnp.float32),
        pltpu.VMEM((RPC,), jnp.float32),
        pltpu.VMEM((RPC, K), jnp.float32),
        pltpu.VMEM((256,), jnp.float32),
        pltpu.VMEM((256,), jnp.float32),
        pltpu.VMEM((C,), jnp.float32),
        pltpu.VMEM((C,), jnp.float32),
        pltpu.SemaphoreType.DMA,
        pltpu.SemaphoreType.DMA,
    ],
    compiler_params=pltpu.CompilerParams(
        use_tc_tiling_on_sc=False, needs_layout_passes=False),
)(_fm_body)


def kernel(feature_idx, feature_values, W_first, W_second):
    idx_flat = feature_idx.reshape(B * F)
    vals_flat = feature_values.reshape(B * F)
    w1_flat = W_first.reshape(-1)
    return _fm(idx_flat, vals_flat, w1_flat, W_second)
